# baseline (speedup 1.0000x reference)
import functools

import jax
import jax.numpy as jnp
from jax import lax
from jax.experimental import pallas as pl
from jax.experimental.pallas import tpu as pltpu
from jax.experimental.pallas import tpu_sc as plsc

N, D = 10000, 128
NC, NS = 2, 16
NW = NC * NS
L = 16
C = 80
NBUF = 4
RB = 624
ZR = 16


def _sc_spmm(row, col, vals, x, wsb):
    E = row.shape[0]
    epw = E // NW
    nch = epw // C
    nout = nch // NBUF

    mesh = plsc.VectorSubcoreMesh(core_axis_name="c", subcore_axis_name="s")

    @functools.partial(
        pl.kernel,
        out_type=jax.ShapeDtypeStruct((NC, N, D), jnp.float32),
        mesh=mesh,
        scratch_types=[
            [pltpu.VMEM((C,), jnp.int32) for _ in range(NBUF)],
            [pltpu.VMEM((C,), jnp.int32) for _ in range(NBUF)],
            [pltpu.VMEM((C,), jnp.float32) for _ in range(NBUF)],
            [pltpu.VMEM((C, D), jnp.float32) for _ in range(NBUF)],
            pltpu.VMEM((ZR, D), jnp.float32),
            pltpu.VMEM((L,), jnp.float32),
            pltpu.VMEM_SHARED((N, D), jnp.float32),
            pltpu.SemaphoreType.DMA((NBUF,)),
            pltpu.SemaphoreType.DMA((NBUF,)),
            pltpu.SemaphoreType.DMA((NBUF,)),
        ],
    )
    def k(row_h, col_h, vals_h, x_h, wsb_h, out_h,
          colm, rowm, valm, rows, zbuf, wsv, acc, msem, gsem, ssem):
        cid = lax.axis_index("c")
        sid = lax.axis_index("s")
        wid = sid * NC + cid
        base = wid * epw

        pltpu.sync_copy(wsb_h, wsv)
        ws_vec = wsv[...]

        for i in range(ZR):
            for j in range(D // L):
                zbuf[i, pl.ds(j * L, L)] = jnp.zeros((L,), jnp.float32)
        nblk = jnp.where(sid == NS - 1, (N - (NS - 1) * RB) // ZR, RB // ZR)

        def zblk(t, _):
            pltpu.sync_copy(zbuf, acc.at[pl.ds(sid * RB + t * ZR, ZR)])
            return 0
        lax.fori_loop(0, nblk, zblk, 0)
        plsc.subcore_barrier()

        def issue_meta(b, kk):
            off = base + kk * C
            pltpu.async_copy(row_h.at[pl.ds(off, C)], rowm[b], msem.at[b])
            pltpu.async_copy(col_h.at[pl.ds(off, C)], colm[b], msem.at[b])
            pltpu.async_copy(vals_h.at[pl.ds(off, C)], valm[b], msem.at[b])

        def wait_meta(b, kk):
            off = base + kk * C
            pltpu.make_async_copy(row_h.at[pl.ds(off, C)], rowm[b],
                                  msem.at[b]).wait()
            pltpu.make_async_copy(col_h.at[pl.ds(off, C)], colm[b],
                                  msem.at[b]).wait()
            pltpu.make_async_copy(vals_h.at[pl.ds(off, C)], valm[b],
                                  msem.at[b]).wait()

        def issue_gather(b):
            pltpu.async_copy(x_h.at[colm[b]], rows[b], gsem.at[b])

        def wait_gather(b):
            pltpu.make_async_copy(x_h.at[colm[b]], rows[b], gsem.at[b]).wait()

        def issue_scatter(b):
            pltpu.async_copy(rows[b], acc.at[rowm[b]], ssem.at[b], add=True)

        def wait_scatter(b):
            pltpu.make_async_copy(rows[b], acc.at[rowm[b]], ssem.at[b]).wait()

        def scale(b):
            def grp(g, _):
                v = valm[b][pl.ds(g * L, L)] * ws_vec
                for t in range(L):
                    sc = v[t]
                    wt = rows[b].at[g * L + t]
                    for j in range(D // L):
                        sl = pl.ds(j * L, L)
                        wt[sl] = wt[sl] * sc
                return 0
            lax.fori_loop(0, C // L, grp, 0)

        for b in range(NBUF):
            issue_meta(b, b)
        for b in range(NBUF):
            wait_meta(b, b)
            issue_gather(b)
        for b in range(NBUF):
            wait_gather(b)
            scale(b)
            issue_scatter(b)

        def round_(r, _):
            for b in range(NBUF):
                wait_scatter(b)
                issue_meta(b, r * NBUF + b)
            for b in range(NBUF):
                wait_meta(b, r * NBUF + b)
                issue_gather(b)
            for b in range(NBUF):
                wait_gather(b)
                scale(b)
                issue_scatter(b)
            return 0
        lax.fori_loop(1, nout, round_, 0)

        for kk in range(nout * NBUF, nch):
            wait_scatter(0)
            issue_meta(0, kk)
            wait_meta(0, kk)
            issue_gather(0)
            wait_gather(0)
            scale(0)
            issue_scatter(0)

        for b in range(NBUF):
            wait_scatter(b)

        plsc.subcore_barrier()

        def oblk(t, _):
            s = sid * RB + t * ZR
            pltpu.sync_copy(acc.at[pl.ds(s, ZR)], out_h.at[cid, pl.ds(s, ZR)])
            return 0
        lax.fori_loop(0, nblk, oblk, 0)

    return k(row, col, vals, x, wsb)


def _combine_body(p_ref, o_ref):
    o_ref[...] = p_ref[0] + p_ref[1]


def _combine(partials):
    blk = 1000
    return pl.pallas_call(
        _combine_body,
        out_shape=jax.ShapeDtypeStruct((N, D), jnp.float32),
        grid=(N // blk,),
        in_specs=[pl.BlockSpec((NC, blk, D), lambda i: (0, i, 0))],
        out_specs=pl.BlockSpec((blk, D), lambda i: (i, 0)),
    )(partials)


def kernel(x, adj_indices, adj_values, ws, idx):
    row = adj_indices[idx, 0]
    col = adj_indices[idx, 1]
    vals = adj_values[idx]
    wsb = jnp.broadcast_to(ws[idx], (L,))
    partials = _sc_spmm(row, col, vals, x, wsb)
    return _combine(partials)


# speedup vs baseline: 1.0955x; 1.0955x over previous
import functools

import jax
import jax.numpy as jnp
from jax import lax
from jax.experimental import pallas as pl
from jax.experimental.pallas import tpu as pltpu
from jax.experimental.pallas import tpu_sc as plsc

N, D = 10000, 128
NC, NS = 2, 16
NW = NC * NS
L = 16
C = 80
NBUF = 4
RB = 624
ZR = 16


def _sc_spmm(af, vf, x, wsb, idxv, E):
    epw = E // NW
    nch = epw // C
    nout = nch // NBUF

    mesh = plsc.VectorSubcoreMesh(core_axis_name="c", subcore_axis_name="s")

    @functools.partial(
        pl.kernel,
        out_type=jax.ShapeDtypeStruct((NC, N, D), jnp.float32),
        mesh=mesh,
        scratch_types=[
            [pltpu.VMEM((C,), jnp.int32) for _ in range(NBUF)],
            [pltpu.VMEM((C,), jnp.int32) for _ in range(NBUF)],
            [pltpu.VMEM((C,), jnp.float32) for _ in range(NBUF)],
            [pltpu.VMEM((C, D), jnp.float32) for _ in range(NBUF)],
            pltpu.VMEM((ZR, D), jnp.float32),
            pltpu.VMEM((L,), jnp.float32),
            pltpu.VMEM((L,), jnp.int32),
            pltpu.VMEM_SHARED((N, D), jnp.float32),
            pltpu.SemaphoreType.DMA((NBUF,)),
            pltpu.SemaphoreType.DMA((NBUF,)),
            pltpu.SemaphoreType.DMA((NBUF,)),
            pltpu.SemaphoreType.DMA,
        ],
    )
    def k(af_h, vf_h, x_h, wsb_h, idxv_h, out_h,
          colm, rowm, valm, rows, zbuf, wsv, idxb, acc, msem, gsem, ssem,
          zsem):
        cid = lax.axis_index("c")
        sid = lax.axis_index("s")
        wid = sid * NC + cid
        base = wid * epw

        pltpu.sync_copy(wsb_h, wsv)
        pltpu.sync_copy(idxv_h, idxb)
        ws_vec = wsv[...]
        i0 = idxb[...][0]
        roff = i0 * (2 * E) + base
        coff = roff + E
        voff = i0 * E + base

        for i in range(ZR):
            for j in range(D // L):
                zbuf[i, pl.ds(j * L, L)] = jnp.zeros((L,), jnp.float32)
        nblk = jnp.where(sid == NS - 1, (N - (NS - 1) * RB) // ZR, RB // ZR)

        def zblk(t, _):
            pltpu.async_copy(zbuf, acc.at[pl.ds(sid * RB + t * ZR, ZR)],
                             zsem)
            return 0
        lax.fori_loop(0, nblk, zblk, 0)

        def zwait(t, _):
            pltpu.make_async_copy(zbuf, acc.at[pl.ds(sid * RB, ZR)],
                                  zsem).wait()
            return 0
        lax.fori_loop(0, nblk, zwait, 0)
        plsc.subcore_barrier()

        def issue_meta(b, kk):
            pltpu.async_copy(af_h.at[pl.ds(roff + kk * C, C)], rowm[b],
                             msem.at[b])
            pltpu.async_copy(af_h.at[pl.ds(coff + kk * C, C)], colm[b],
                             msem.at[b])
            pltpu.async_copy(vf_h.at[pl.ds(voff + kk * C, C)], valm[b],
                             msem.at[b])

        def wait_meta(b, kk):
            pltpu.make_async_copy(af_h.at[pl.ds(roff + kk * C, C)], rowm[b],
                                  msem.at[b]).wait()
            pltpu.make_async_copy(af_h.at[pl.ds(coff + kk * C, C)], colm[b],
                                  msem.at[b]).wait()
            pltpu.make_async_copy(vf_h.at[pl.ds(voff + kk * C, C)], valm[b],
                                  msem.at[b]).wait()

        def issue_gather(b):
            pltpu.async_copy(x_h.at[colm[b]], rows[b], gsem.at[b])

        def wait_gather(b):
            pltpu.make_async_copy(x_h.at[colm[b]], rows[b], gsem.at[b]).wait()

        def issue_scatter(b):
            pltpu.async_copy(rows[b], acc.at[rowm[b]], ssem.at[b], add=True)

        def wait_scatter(b):
            pltpu.make_async_copy(rows[b], acc.at[rowm[b]], ssem.at[b]).wait()

        def scale(b):
            def grp(g, _):
                v = valm[b][pl.ds(g * L, L)] * ws_vec
                for t in range(L):
                    sc = v[t]
                    wt = rows[b].at[g * L + t]
                    for j in range(D // L):
                        sl = pl.ds(j * L, L)
                        wt[sl] = wt[sl] * sc
                return 0
            lax.fori_loop(0, C // L, grp, 0)

        for b in range(NBUF):
            issue_meta(b, b)
        for b in range(NBUF):
            wait_meta(b, b)
            issue_gather(b)
        for b in range(NBUF):
            wait_gather(b)
            scale(b)
            issue_scatter(b)

        def round_(r, _):
            for b in range(NBUF):
                wait_scatter(b)
                issue_meta(b, r * NBUF + b)
            for b in range(NBUF):
                wait_meta(b, r * NBUF + b)
                issue_gather(b)
            for b in range(NBUF):
                wait_gather(b)
                scale(b)
                issue_scatter(b)
            return 0
        lax.fori_loop(1, nout, round_, 0)

        for kk in range(nout * NBUF, nch):
            wait_scatter(0)
            issue_meta(0, kk)
            wait_meta(0, kk)
            issue_gather(0)
            wait_gather(0)
            scale(0)
            issue_scatter(0)

        for b in range(NBUF):
            wait_scatter(b)

        plsc.subcore_barrier()
        pltpu.sync_copy(acc.at[pl.ds(sid * RB, RB)],
                        out_h.at[cid, pl.ds(sid * RB, RB)])

        @pl.when(sid == NS - 1)
        def _():
            pltpu.sync_copy(acc.at[pl.ds((NS - 1) * RB + RB, N - NS * RB)],
                            out_h.at[cid, pl.ds(NS * RB, N - NS * RB)])

    return k(af, vf, x, wsb, idxv)


def _combine_body(p_ref, o_ref):
    o_ref[...] = p_ref[0] + p_ref[1]


def _combine(partials):
    blk = 1000
    return pl.pallas_call(
        _combine_body,
        out_shape=jax.ShapeDtypeStruct((N, D), jnp.float32),
        grid=(N // blk,),
        in_specs=[pl.BlockSpec((NC, blk, D), lambda i: (0, i, 0))],
        out_specs=pl.BlockSpec((blk, D), lambda i: (i, 0)),
    )(partials)


def kernel(x, adj_indices, adj_values, ws, idx):
    E = adj_values.shape[1]
    af = adj_indices.reshape(-1)
    vf = adj_values.reshape(-1)
    idxv = jnp.full((L,), idx, jnp.int32)
    wsb = jnp.broadcast_to(ws[idx], (L,))
    partials = _sc_spmm(af, vf, x, wsb, idxv, E)
    return _combine(partials)


# speedup vs baseline: 1.1241x; 1.0261x over previous
import functools

import jax
import jax.numpy as jnp
from jax import lax
from jax.experimental import pallas as pl
from jax.experimental.pallas import tpu as pltpu
from jax.experimental.pallas import tpu_sc as plsc

N, D = 10000, 128
NC, NS = 2, 16
NW = NC * NS
L = 16
C = 80
NBUF = 4
RB = 624
ZR = 16


def _sc_spmm(af, vf, x, wsb, E):
    epw = E // NW
    nch = epw // C
    nout = nch // NBUF

    mesh = plsc.VectorSubcoreMesh(core_axis_name="c", subcore_axis_name="s")

    @functools.partial(
        pl.kernel,
        out_type=jax.ShapeDtypeStruct((NC, N, D), jnp.float32),
        mesh=mesh,
        scratch_types=[
            [pltpu.VMEM((C,), jnp.int32) for _ in range(NBUF)],
            [pltpu.VMEM((C,), jnp.int32) for _ in range(NBUF)],
            [pltpu.VMEM((C,), jnp.float32) for _ in range(NBUF)],
            [pltpu.VMEM((C, D), jnp.float32) for _ in range(NBUF)],
            pltpu.VMEM((ZR, D), jnp.float32),
            pltpu.VMEM((L,), jnp.float32),
            pltpu.VMEM_SHARED((N, D), jnp.float32),
            pltpu.SemaphoreType.DMA((NBUF,)),
            pltpu.SemaphoreType.DMA((NBUF,)),
            pltpu.SemaphoreType.DMA((NBUF,)),
            pltpu.SemaphoreType.DMA,
        ],
    )
    def k(af_h, vf_h, x_h, wsb_h, out_h,
          colm, rowm, valm, rows, zbuf, wsv, acc, msem, gsem, ssem,
          zsem):
        cid = lax.axis_index("c")
        sid = lax.axis_index("s")
        wid = sid * NC + cid
        base = wid * epw

        pltpu.sync_copy(wsb_h, wsv)
        ws_vec = wsv[...]
        roff = base
        coff = E + base
        voff = base

        for i in range(ZR):
            for j in range(D // L):
                zbuf[i, pl.ds(j * L, L)] = jnp.zeros((L,), jnp.float32)
        nblk = jnp.where(sid == NS - 1, (N - (NS - 1) * RB) // ZR, RB // ZR)

        def zblk(t, _):
            pltpu.async_copy(zbuf, acc.at[pl.ds(sid * RB + t * ZR, ZR)],
                             zsem)
            return 0
        lax.fori_loop(0, nblk, zblk, 0)

        def zwait(t, _):
            pltpu.make_async_copy(zbuf, acc.at[pl.ds(sid * RB, ZR)],
                                  zsem).wait()
            return 0
        lax.fori_loop(0, nblk, zwait, 0)
        plsc.subcore_barrier()

        def issue_meta(b, kk):
            pltpu.async_copy(af_h.at[pl.ds(roff + kk * C, C)], rowm[b],
                             msem.at[b])
            pltpu.async_copy(af_h.at[pl.ds(coff + kk * C, C)], colm[b],
                             msem.at[b])
            pltpu.async_copy(vf_h.at[pl.ds(voff + kk * C, C)], valm[b],
                             msem.at[b])

        def wait_meta(b, kk):
            pltpu.make_async_copy(af_h.at[pl.ds(roff + kk * C, C)], rowm[b],
                                  msem.at[b]).wait()
            pltpu.make_async_copy(af_h.at[pl.ds(coff + kk * C, C)], colm[b],
                                  msem.at[b]).wait()
            pltpu.make_async_copy(vf_h.at[pl.ds(voff + kk * C, C)], valm[b],
                                  msem.at[b]).wait()

        def issue_gather(b):
            pltpu.async_copy(x_h.at[colm[b]], rows[b], gsem.at[b])

        def wait_gather(b):
            pltpu.make_async_copy(x_h.at[colm[b]], rows[b], gsem.at[b]).wait()

        def issue_scatter(b):
            pltpu.async_copy(rows[b], acc.at[rowm[b]], ssem.at[b], add=True)

        def wait_scatter(b):
            pltpu.make_async_copy(rows[b], acc.at[rowm[b]], ssem.at[b]).wait()

        def scale(b):
            def grp(g, _):
                v = valm[b][pl.ds(g * L, L)] * ws_vec
                for t in range(L):
                    sc = v[t]
                    wt = rows[b].at[g * L + t]
                    for j in range(D // L):
                        sl = pl.ds(j * L, L)
                        wt[sl] = wt[sl] * sc
                return 0
            lax.fori_loop(0, C // L, grp, 0)

        for b in range(NBUF):
            issue_meta(b, b)
        for b in range(NBUF):
            wait_meta(b, b)
            issue_gather(b)
        for b in range(NBUF):
            wait_gather(b)
            scale(b)
            issue_scatter(b)

        def round_(r, _):
            for b in range(NBUF):
                wait_scatter(b)
                issue_meta(b, r * NBUF + b)
            for b in range(NBUF):
                wait_meta(b, r * NBUF + b)
                issue_gather(b)
            for b in range(NBUF):
                wait_gather(b)
                scale(b)
                issue_scatter(b)
            return 0
        lax.fori_loop(1, nout, round_, 0)

        for kk in range(nout * NBUF, nch):
            wait_scatter(0)
            issue_meta(0, kk)
            wait_meta(0, kk)
            issue_gather(0)
            wait_gather(0)
            scale(0)
            issue_scatter(0)

        for b in range(NBUF):
            wait_scatter(b)

        plsc.subcore_barrier()
        pltpu.sync_copy(acc.at[pl.ds(sid * RB, RB)],
                        out_h.at[cid, pl.ds(sid * RB, RB)])

        @pl.when(sid == NS - 1)
        def _():
            pltpu.sync_copy(acc.at[pl.ds((NS - 1) * RB + RB, N - NS * RB)],
                            out_h.at[cid, pl.ds(NS * RB, N - NS * RB)])

    return k(af, vf, x, wsb)


def _combine_body(p_ref, o_ref):
    o_ref[...] = p_ref[0] + p_ref[1]


def _combine(partials):
    blk = 1000
    return pl.pallas_call(
        _combine_body,
        out_shape=jax.ShapeDtypeStruct((N, D), jnp.float32),
        grid=(N // blk,),
        in_specs=[pl.BlockSpec((NC, blk, D), lambda i: (0, i, 0))],
        out_specs=pl.BlockSpec((blk, D), lambda i: (i, 0)),
    )(partials)


def kernel(x, adj_indices, adj_values, ws, idx):
    E = adj_values.shape[1]
    af = adj_indices[idx].reshape(-1)
    vf = adj_values[idx]
    wsb = jnp.broadcast_to(ws[idx], (L,))
    partials = _sc_spmm(af, vf, x, wsb, E)
    return _combine(partials)


# speedup vs baseline: 1.1401x; 1.0143x over previous
import functools

import jax
import jax.numpy as jnp
from jax import lax
from jax.experimental import pallas as pl
from jax.experimental.pallas import tpu as pltpu
from jax.experimental.pallas import tpu_sc as plsc

N, D = 10000, 128
NC, NS = 2, 16
NW = NC * NS
L = 16
C = 80
NBUF = 2
RB = 624


def _sc_spmm(af, vf, x, wsb, E):
    nch = (E // NW) // C
    nout = nch // NBUF

    mesh = plsc.VectorSubcoreMesh(core_axis_name="c", subcore_axis_name="s")

    @functools.partial(
        pl.kernel,
        out_type=jax.ShapeDtypeStruct((NC, N, D), jnp.float32),
        mesh=mesh,
        scratch_types=[
            pltpu.VMEM((nch * C,), jnp.int32),
            pltpu.VMEM((nch * C,), jnp.int32),
            [pltpu.VMEM((C,), jnp.float32) for _ in range(NBUF)],
            [pltpu.VMEM((C,), jnp.int32) for _ in range(NBUF)],
            [pltpu.VMEM((C, D), jnp.float32) for _ in range(NBUF)],
            pltpu.VMEM((L,), jnp.float32),
            pltpu.VMEM_SHARED((N, D), jnp.float32),
            pltpu.SemaphoreType.DMA((NBUF,)),
            pltpu.SemaphoreType.DMA((NBUF,)),
            pltpu.SemaphoreType.DMA,
        ],
    )
    def k(af_h, vf_h, x_h, wsb_h, out_h,
          colm1, rowm1, valm, rowm, rows, wsv, acc, gsem, ssem, zsem):
        cid = lax.axis_index("c")
        sid = lax.axis_index("s")
        wid = sid * NC + cid
        epw = nch * C
        base = wid * epw

        pltpu.async_copy(af_h.at[pl.ds(base, epw)], rowm1, zsem)
        pltpu.async_copy(af_h.at[pl.ds(E + base, epw)], colm1, zsem)
        pltpu.sync_copy(wsb_h, wsv)
        ws_vec = wsv[...]

        def zrow(i, _):
            for j in range(D // L):
                rows[0][i, pl.ds(j * L, L)] = jnp.zeros((L,), jnp.float32)
            return 0
        lax.fori_loop(0, C, zrow, 0)
        for t in range(RB // C):
            pltpu.async_copy(rows[0], acc.at[pl.ds(sid * RB + t * C, C)],
                             zsem)
        pltpu.async_copy(rows[0].at[pl.ds(0, RB - (RB // C) * C)],
                         acc.at[pl.ds(sid * RB + (RB // C) * C,
                                      RB - (RB // C) * C)], zsem)

        @pl.when(sid == NS - 1)
        def _():
            pltpu.async_copy(rows[0].at[pl.ds(0, N - NS * RB)],
                             acc.at[pl.ds(NS * RB, N - NS * RB)], zsem)

        pltpu.make_async_copy(af_h.at[pl.ds(base, epw)], rowm1, zsem).wait()
        pltpu.make_async_copy(af_h.at[pl.ds(E + base, epw)], colm1,
                              zsem).wait()
        for t in range(RB // C):
            pltpu.make_async_copy(rows[0],
                                  acc.at[pl.ds(sid * RB, C)], zsem).wait()
        pltpu.make_async_copy(rows[0].at[pl.ds(0, RB - (RB // C) * C)],
                              acc.at[pl.ds(sid * RB,
                                           RB - (RB // C) * C)], zsem).wait()

        @pl.when(sid == NS - 1)
        def _():
            pltpu.make_async_copy(rows[0].at[pl.ds(0, N - NS * RB)],
                                  acc.at[pl.ds(NS * RB, N - NS * RB)],
                                  zsem).wait()
        plsc.subcore_barrier()

        def issue_gather(b, kk):
            for g in range(C // L):
                rowm[b][pl.ds(g * L, L)] = rowm1[pl.ds(kk * C + g * L, L)]
            pltpu.async_copy(vf_h.at[pl.ds(base + kk * C, C)], valm[b],
                             gsem.at[b])
            pltpu.async_copy(x_h.at[colm1.at[pl.ds(kk * C, C)]], rows[b],
                             gsem.at[b])

        def wait_gather(b, kk):
            pltpu.make_async_copy(vf_h.at[pl.ds(base + kk * C, C)], valm[b],
                                  gsem.at[b]).wait()
            pltpu.make_async_copy(x_h.at[colm1.at[pl.ds(kk * C, C)]],
                                  rows[b], gsem.at[b]).wait()

        def issue_scatter(b, kk):
            pltpu.async_copy(rows[b], acc.at[rowm[b]], ssem.at[b],
                             add=True)

        def wait_scatter(b, kk):
            pltpu.make_async_copy(rows[b], acc.at[rowm[b]],
                                  ssem.at[b]).wait()

        def scale(b, kk):
            def grp(g, _):
                v = valm[b][pl.ds(g * L, L)] * ws_vec
                for t in range(L):
                    sc = v[t]
                    wt = rows[b].at[g * L + t]
                    for j in range(D // L):
                        sl = pl.ds(j * L, L)
                        wt[sl] = wt[sl] * sc
                return 0
            lax.fori_loop(0, C // L, grp, 0)

        for b in range(NBUF):
            issue_gather(b, b)
        for b in range(NBUF):
            wait_gather(b, b)
            scale(b, b)
            issue_scatter(b, b)

        def round_(r, _):
            kks = [r * NBUF + b for b in range(NBUF)]
            for b in range(NBUF):
                wait_scatter(b, kks[b] - NBUF)
                issue_gather(b, kks[b])
            for b in range(NBUF):
                wait_gather(b, kks[b])
                scale(b, kks[b])
                issue_scatter(b, kks[b])
            return 0
        lax.fori_loop(1, nout, round_, 0)

        for kk in range(nout * NBUF, nch):
            wait_scatter(0, kk - NBUF)
            issue_gather(0, kk)
            wait_gather(0, kk)
            scale(0, kk)
            issue_scatter(0, kk)

        for b in range(NBUF):
            wait_scatter(b, nch - NBUF + b)

        plsc.subcore_barrier()
        pltpu.sync_copy(acc.at[pl.ds(sid * RB, RB)],
                        out_h.at[cid, pl.ds(sid * RB, RB)])

        @pl.when(sid == NS - 1)
        def _():
            pltpu.sync_copy(acc.at[pl.ds(NS * RB, N - NS * RB)],
                            out_h.at[cid, pl.ds(NS * RB, N - NS * RB)])

    return k(af, vf, x, wsb)


def _combine_body(p_ref, o_ref):
    o_ref[...] = p_ref[0] + p_ref[1]


def _combine(partials):
    blk = 1000
    return pl.pallas_call(
        _combine_body,
        out_shape=jax.ShapeDtypeStruct((N, D), jnp.float32),
        grid=(N // blk,),
        in_specs=[pl.BlockSpec((NC, blk, D), lambda i: (0, i, 0))],
        out_specs=pl.BlockSpec((blk, D), lambda i: (i, 0)),
    )(partials)


def kernel(x, adj_indices, adj_values, ws, idx):
    E = adj_values.shape[1]
    af = adj_indices[idx].reshape(-1)
    vf = adj_values[idx]
    wsb = jnp.broadcast_to(ws[idx], (L,))
    partials = _sc_spmm(af, vf, x, wsb, E)
    return _combine(partials)
